# baseline (speedup 1.0000x reference)
import dataclasses
import functools

import jax
import jax.numpy as jnp
from jax import lax
from jax.experimental import pallas as pl
from jax.experimental.pallas import tpu as pltpu
from jax.experimental.pallas import tpu_sc as plsc

N = 10000
E = 320000
HD = 128
NH = 8
DH = 16
R = 100
INV_SCALE = 0.25

NC = 2
NS = 16
NW = NC * NS
EPT = E // NW
BLK = 80
NBLK = EPT // BLK
ZCH = 80
NZC = N // ZCH
ZPT = -(-NZC // NS)


ND = 640
DPT = ND // NS
NBH0 = 63


def _edge_pass(src3, dst3, typ3, q2, k2, v2, rel1, zn, zd):
    mesh = plsc.VectorSubcoreMesh(core_axis_name="c", subcore_axis_name="s")
    cp = pltpu.CompilerParams()
    if "needs_layout_passes" in pltpu.CompilerParams.__dataclass_fields__:
        cp = dataclasses.replace(cp, needs_layout_passes=False)
    if "use_tc_tiling_on_sc" in pltpu.CompilerParams.__dataclass_fields__:
        cp = dataclasses.replace(cp, use_tc_tiling_on_sc=False)

    @functools.partial(
        pl.kernel,
        compiler_params=cp,
        out_type=[jax.ShapeDtypeStruct((NC, N, HD), jnp.bfloat16),
                  jax.ShapeDtypeStruct((NC, ND, HD), jnp.bfloat16)],
        mesh=mesh,
        scratch_types=[
            pltpu.VMEM((NBH0, BLK), jnp.int32),
            pltpu.VMEM((NBH0, BLK), jnp.int32),
            pltpu.VMEM((NBH0, BLK), jnp.int32),
            pltpu.VMEM((R * HD,), jnp.bfloat16),
            pltpu.VMEM((2 * DH,), jnp.float32),
            pltpu.VMEM((BLK * DH,), jnp.float32),
            pltpu.VMEM((2, BLK, HD), jnp.bfloat16),
            pltpu.VMEM((2, BLK, HD), jnp.bfloat16),
            pltpu.VMEM((2, BLK, HD), jnp.bfloat16),
            pltpu.VMEM((2, BLK, HD), jnp.bfloat16),
            pltpu.VMEM((2, BLK, HD), jnp.bfloat16),
            pltpu.VMEM((2, BLK), jnp.int32),
            pltpu.VMEM_SHARED((N, HD), jnp.bfloat16),
            pltpu.VMEM_SHARED((ND, HD), jnp.bfloat16),
            pltpu.SemaphoreType.DMA,
            pltpu.SemaphoreType.DMA,
            pltpu.SemaphoreType.DMA,
            pltpu.SemaphoreType.DMA,
        ],
    )
    def k(src_h, dst_h, typ_h, q_h, k_h, v_h, rel_h, zn_h, zd_h, outn_h, outd_h,
          srci, dsti, typi, relv, swp, evb, qg, kg, vg, stgn, stgd, drow,
          accn, accd, gs0, gs1, ss0, ss1):
        cid = lax.axis_index("c")
        sid = lax.axis_index("s")
        wid = cid * NS + sid
        gsem = (gs0, gs1)
        ssem = (ss0, ss1)

        pltpu.sync_copy(rel_h, relv)

        for t in range(ZPT):
            c = sid + t * NS

            @pl.when(c < NZC)
            def _():
                pltpu.sync_copy(zn_h, accn.at[pl.ds(c * ZCH, ZCH)])

        pltpu.sync_copy(zd_h, accd.at[pl.ds(sid * DPT, DPT)])
        plsc.subcore_barrier()

        zero = jnp.zeros((DH,), jnp.float32)
        lane = lax.broadcasted_iota(jnp.int32, (DH,), 0)
        lane7 = lane & 7
        lhalf = lane >> 3

        def issue_gathers(b, u):
            pltpu.async_copy(q_h.at[dsti.at[b]], qg.at[u], gsem[u])
            pltpu.async_copy(k_h.at[srci.at[b]], kg.at[u], gsem[u])
            pltpu.async_copy(v_h.at[srci.at[b]], vg.at[u], gsem[u])

        def drain_gathers(b, u):
            pltpu.make_async_copy(q_h.at[dsti.at[b]], qg.at[u], gsem[u]).wait()
            pltpu.make_async_copy(k_h.at[srci.at[b]], kg.at[u], gsem[u]).wait()
            pltpu.make_async_copy(v_h.at[srci.at[b]], vg.at[u], gsem[u]).wait()

        def issue_scatters(b, u):
            pltpu.async_copy(stgn.at[u], accn.at[dsti.at[b]], ssem[u], add=True)
            pltpu.async_copy(stgd.at[u], accd.at[drow.at[u]], ssem[u], add=True)

        def drain_scatters(b, u):
            pltpu.make_async_copy(stgn.at[u], accn.at[dsti.at[b]],
                                  ssem[u]).wait()
            pltpu.make_async_copy(stgd.at[u], accd.at[drow.at[u]],
                                  ssem[u]).wait()

        zb32 = jnp.zeros((2 * DH,), jnp.bfloat16)

        def compute(b, u):
            @pl.loop(0, BLK // DH)
            def _(g):
                tv = typi[b, pl.ds(g * DH, DH)]
                for j in range(DH):
                    e = g * DH + j
                    rb = tv[j] * HD
                    acc0 = zero
                    acc1 = zero
                    for t in range(NH // 2):
                        qa, qb = plsc.unpack(
                            qg[u, e, pl.ds(t * 2 * DH, 2 * DH)],
                            format=plsc.PackFormat.INTERLEAVED)
                        ka, kb = plsc.unpack(
                            kg[u, e, pl.ds(t * 2 * DH, 2 * DH)],
                            format=plsc.PackFormat.INTERLEAVED)
                        ra, rc = plsc.unpack(
                            relv[pl.ds(rb + t * 2 * DH, 2 * DH)],
                            format=plsc.PackFormat.INTERLEAVED)
                        acc0 = acc0 + qa * (ka + ra)
                        acc1 = acc1 + qb * (kb + rc)
                    acc = acc0 + acc1
                    swp[pl.ds(0, DH)] = acc
                    swp[pl.ds(DH, DH)] = acc
                    sv = acc + swp[pl.ds(NH, DH)]
                    evb[pl.ds(e * DH, DH)] = jnp.exp(
                        jnp.minimum(sv * INV_SCALE, 60.0))

            @pl.loop(0, BLK // DH)
            def _(g):
                dv = dsti[b, pl.ds(g * DH, DH)]
                drow[u, pl.ds(g * DH, DH)] = dv >> 4
                for j in range(DH):
                    e = g * DH + j
                    d = dv[j]
                    ev = evb[pl.ds(e * DH, DH)]
                    for i in range(NH // 2):
                        esc = plsc.pack(zero + ev[2 * i], zero + ev[2 * i + 1],
                                        format=plsc.PackFormat.INTERLEAVED)
                        stgn[u, e, pl.ds(i * 2 * DH, 2 * DH)] = (
                            vg[u, e, pl.ds(i * 2 * DH, 2 * DH)] * esc)
                    dmask = jnp.where(lhalf == (d & 1), 1.0, 0.0)
                    dvreg = ev * dmask
                    p = (d & 15) >> 1
                    pp = p >> 1
                    packed = plsc.pack(
                        jnp.where((p & 1) == 0, dvreg, zero),
                        jnp.where((p & 1) == 1, dvreg, zero),
                        format=plsc.PackFormat.INTERLEAVED)
                    for i in range(NH // 2):
                        stgd[u, e, pl.ds(i * 2 * DH, 2 * DH)] = jnp.where(
                            pp == i, packed, zb32)

        def run_half(base, nbh):
            rows = pl.ds(0, nbh)
            pltpu.sync_copy(src_h.at[wid, pl.ds(base, nbh)], srci.at[rows])
            pltpu.sync_copy(dst_h.at[wid, pl.ds(base, nbh)], dsti.at[rows])
            pltpu.sync_copy(typ_h.at[wid, pl.ds(base, nbh)], typi.at[rows])
            issue_gathers(0, 0)

            @pl.loop(0, nbh // 2)
            def _(pr):
                b0 = 2 * pr
                b1 = 2 * pr + 1
                drain_gathers(b0, 0)
                issue_gathers(b1, 1)

                @pl.when(pr > 0)
                def _():
                    drain_scatters(b0 - 2, 0)

                compute(b0, 0)
                issue_scatters(b0, 0)

                drain_gathers(b1, 1)

                @pl.when(b1 + 1 < nbh)
                def _():
                    issue_gathers(b1 + 1, 0)

                @pl.when(pr > 0)
                def _():
                    drain_scatters(b1 - 2, 1)

                compute(b1, 1)
                issue_scatters(b1, 1)

            if nbh % 2:
                tb = nbh - 1
                drain_gathers(tb, 0)
                drain_scatters(tb - 2, 0)
                compute(tb, 0)
                issue_scatters(tb, 0)
                drain_scatters(tb, 0)
                drain_scatters(tb - 1, 1)
            else:
                drain_scatters(nbh - 2, 0)
                drain_scatters(nbh - 1, 1)

        run_half(0, NBH0)
        run_half(NBH0, NBLK - NBH0)

        plsc.subcore_barrier()
        for t in range(ZPT):
            c = sid + t * NS

            @pl.when(c < NZC)
            def _():
                rows = pl.ds(c * ZCH, ZCH)
                pltpu.sync_copy(accn.at[rows], outn_h.at[cid].at[rows])

        drows = pl.ds(sid * DPT, DPT)
        pltpu.sync_copy(accd.at[drows], outd_h.at[cid].at[drows])

    return k(src3, dst3, typ3, q2, k2, v2, rel1, zn, zd)


def _qkv(nf, qe, rel, wqt, wqb, bq, wkt, wkb, bk, wv, bv):
    def body(nf_r, qe_r, rel_r, wqt_r, wqb_r, bq_r, wkt_r, wkb_r, bk_r,
             wv_r, bv_r, q_r, k_r, v_r, rel_o):
        nfv = nf_r[...]
        qev = qe_r[...]
        oi = lax.broadcasted_iota(jnp.int32, (HD, HD), 0)
        ci = lax.broadcasted_iota(jnp.int32, (HD, HD), 1)
        vsrc = (2 * (ci // 32) + (ci % 2)) * DH + (ci % 32) // 2
        perm_v = jnp.where(oi == vsrc, 1.0, 0.0).astype(jnp.float32)
        t_ = ci // 32
        m_ = (ci % 32) // 2
        par_ = ci % 2
        qksrc = (m_ % NH) * DH + NH * par_ + 2 * t_ + m_ // NH
        perm_qk = jnp.where(oi == qksrc, 1.0, 0.0).astype(jnp.float32)
        qb = jnp.dot(qev, wqb_r[...], preferred_element_type=jnp.float32) + bq_r[...]
        kb = jnp.dot(qev, wkb_r[...], preferred_element_type=jnp.float32) + bk_r[...]
        q = jnp.dot(nfv, wqt_r[...], preferred_element_type=jnp.float32) + qb
        k = jnp.dot(nfv, wkt_r[...], preferred_element_type=jnp.float32) + kb
        v = jnp.dot(nfv, wv_r[...], preferred_element_type=jnp.float32) + bv_r[...]
        q_r[...] = jnp.dot(q, perm_qk, preferred_element_type=jnp.float32).astype(jnp.bfloat16)
        k_r[...] = jnp.dot(k, perm_qk, preferred_element_type=jnp.float32).astype(jnp.bfloat16)
        v_r[...] = jnp.dot(v, perm_v, preferred_element_type=jnp.float32).astype(jnp.bfloat16)
        rel_o[...] = jnp.dot(rel_r[...], perm_qk,
                             preferred_element_type=jnp.float32).astype(jnp.bfloat16)

    return pl.pallas_call(
        body,
        out_shape=[jax.ShapeDtypeStruct((N, HD), jnp.bfloat16)] * 3
        + [jax.ShapeDtypeStruct((R, HD), jnp.bfloat16)],
    )(nf, qe, rel, wqt, wqb, bq, wkt, wkb, bk, wv, bv)


def _layernorm(x, g, b, eps=1e-5):
    m = jnp.mean(x, axis=-1, keepdims=True)
    v = jnp.mean((x - m) * (x - m), axis=-1, keepdims=True)
    return (x - m) / jnp.sqrt(v + eps) * g + b


def _post(nf, num0, num1, den0, den1, Wo, bo, ln1_g, ln1_b,
          Wf1, bf1, Wf2, bf2, ln2_g, ln2_b):
    RB = 2000
    G = N // RB

    def body(nf_r, n0_r, n1_r, d0_r, d1_r, wo_r, bo_r, g1_r, b1_r,
             wf1_r, bf1_r, wf2_r, bf2_r, g2_r, b2_r, out_r):
        nsum = (n0_r[...].astype(jnp.float32) + n1_r[...].astype(jnp.float32))
        ci = lax.broadcasted_iota(jnp.int32, (HD, HD), 0)
        oi = lax.broadcasted_iota(jnp.int32, (HD, HD), 1)
        o_of_c = (2 * (ci // 32) + (ci % 2)) * DH + (ci % 32) // 2
        perm = jnp.where(o_of_c == oi, 1.0, 0.0).astype(jnp.float32)
        num = jnp.dot(nsum, perm, preferred_element_type=jnp.float32)
        den = d0_r[...] + d1_r[...] + 1e-8
        col = lax.broadcasted_iota(jnp.int32, (NH, HD), 1)
        row = lax.broadcasted_iota(jnp.int32, (NH, HD), 0)
        expand = jnp.where(col // DH == row, 1.0, 0.0).astype(jnp.float32)
        denr = jnp.dot(den, expand, preferred_element_type=jnp.float32)
        agg = num / denr
        attn = jnp.dot(agg, wo_r[...], preferred_element_type=jnp.float32) + bo_r[...]
        x = _layernorm(nf_r[...] + attn, g1_r[...], b1_r[...])
        hpre = jnp.dot(x, wf1_r[...], preferred_element_type=jnp.float32) + bf1_r[...]
        hact = 0.5 * hpre * (1.0 + lax.erf(hpre * (2.0 ** -0.5)))
        h2 = jnp.dot(hact, wf2_r[...], preferred_element_type=jnp.float32) + bf2_r[...]
        out_r[...] = _layernorm(x + h2, g2_r[...], b2_r[...])

    rowspec = lambda w: pl.BlockSpec((RB, w), lambda i: (i, 0))
    full = lambda a: pl.BlockSpec(a.shape, lambda i: tuple(0 for _ in a.shape))
    return pl.pallas_call(
        body,
        grid=(G,),
        in_specs=[rowspec(HD), rowspec(HD), rowspec(HD), rowspec(NH), rowspec(NH),
                  full(Wo), full(bo), full(ln1_g), full(ln1_b),
                  full(Wf1), full(bf1), full(Wf2), full(bf2),
                  full(ln2_g), full(ln2_b)],
        out_specs=pl.BlockSpec((RB, HD), lambda i: (i, 0)),
        out_shape=jax.ShapeDtypeStruct((N, HD), jnp.float32),
    )(nf, num0, num1, den0, den1, Wo, bo, ln1_g, ln1_b,
      Wf1, bf1, Wf2, bf2, ln2_g, ln2_b)


def _rel_update(rel, Wh2t, bh2t, Wt2h, bt2h, Wh2h, bh2h, Wt2t, bt2t,
                Wc1, Wc2, Wc3, Wc4, bc, lnr_g, lnr_b):
    def body(rel_r, wa_r, ba_r, wb_r, bb_r, wc_r, bcc_r, wd_r, bd_r,
             w1_r, w2_r, w3_r, w4_r, bc_r, g_r, b_r, out_r):
        r = rel_r[...]
        i1 = jnp.dot(r, wa_r[...], preferred_element_type=jnp.float32) + ba_r[...]
        i2 = jnp.dot(r, wb_r[...], preferred_element_type=jnp.float32) + bb_r[...]
        i3 = jnp.dot(r, wc_r[...], preferred_element_type=jnp.float32) + bcc_r[...]
        i4 = jnp.dot(r, wd_r[...], preferred_element_type=jnp.float32) + bd_r[...]
        comb = (jnp.dot(i1, w1_r[...], preferred_element_type=jnp.float32)
                + jnp.dot(i2, w2_r[...], preferred_element_type=jnp.float32)
                + jnp.dot(i3, w3_r[...], preferred_element_type=jnp.float32)
                + jnp.dot(i4, w4_r[...], preferred_element_type=jnp.float32)
                + bc_r[...])
        out_r[...] = _layernorm(r + comb, g_r[...], b_r[...])

    return pl.pallas_call(
        body,
        out_shape=jax.ShapeDtypeStruct((R, HD), jnp.float32),
    )(rel, Wh2t, bh2t, Wt2h, bt2h, Wh2h, bh2h, Wt2t, bt2t,
      Wc1, Wc2, Wc3, Wc4, bc, lnr_g, lnr_b)


def kernel(node_features, query_embedding, edge_index, edge_type,
           relation_embeddings, Wq, bq, Wk, bk, Wv, bv, Wo, bo, ln1_g, ln1_b,
           Wf1, bf1, Wf2, bf2, ln2_g, ln2_b, Wh2t, bh2t, Wt2h, bt2h,
           Wh2h, bh2h, Wt2t, bt2t, Wc, bc, lnr_g, lnr_b):
    r1 = lambda a: a.reshape(1, -1)
    q, k, v, reli = _qkv(node_features, query_embedding, relation_embeddings,
                         Wq[:HD], Wq[HD:], r1(bq), Wk[:HD], Wk[HD:], r1(bk),
                         Wv, r1(bv))

    part_n, part_d = _edge_pass(
        edge_index[0].reshape(NW, NBLK, BLK),
        edge_index[1].reshape(NW, NBLK, BLK),
        edge_type.reshape(NW, NBLK, BLK),
        q, k, v,
        reli.reshape(R * HD),
        jnp.zeros((ZCH, HD), jnp.bfloat16),
        jnp.zeros((DPT, HD), jnp.bfloat16))

    dg = part_d.astype(jnp.float32).reshape(NC, ND, HD // 32, 16, 2)
    dg = dg.transpose(0, 1, 2, 4, 3).reshape(NC, ND * DH, NH)
    den = dg[:, :N]
    x = _post(node_features, part_n[0], part_n[1], den[0], den[1],
              Wo, r1(bo), r1(ln1_g), r1(ln1_b),
              Wf1, r1(bf1), Wf2, r1(bf2), r1(ln2_g), r1(ln2_b))

    rel = _rel_update(relation_embeddings, Wh2t, r1(bh2t), Wt2h, r1(bt2h),
                      Wh2h, r1(bh2h), Wt2t, r1(bt2t),
                      Wc[:HD], Wc[HD:2 * HD], Wc[2 * HD:3 * HD], Wc[3 * HD:],
                      r1(bc), r1(lnr_g), r1(lnr_b))
    return (x, rel)


# speedup vs baseline: 1.1807x; 1.1807x over previous
import dataclasses
import functools

import jax
import jax.numpy as jnp
from jax import lax
from jax.experimental import pallas as pl
from jax.experimental.pallas import tpu as pltpu
from jax.experimental.pallas import tpu_sc as plsc

N = 10000
E = 320000
HD = 128
NH = 8
DH = 16
R = 100
INV_SCALE = 0.25

NC = 2
NS = 16
NW = NC * NS
EPT = E // NW
BLK = 80
NBLK = EPT // BLK
ZCH = 80
NZC = N // ZCH
ZPT = -(-NZC // NS)


ND = 640
DPT = ND // NS


def _edge_pass(src3, dst3, typ3, q2, k2, v2, rel1, zn, zd):
    mesh = plsc.VectorSubcoreMesh(core_axis_name="c", subcore_axis_name="s")
    cp = pltpu.CompilerParams()
    if "needs_layout_passes" in pltpu.CompilerParams.__dataclass_fields__:
        cp = dataclasses.replace(cp, needs_layout_passes=False)
    if "use_tc_tiling_on_sc" in pltpu.CompilerParams.__dataclass_fields__:
        cp = dataclasses.replace(cp, use_tc_tiling_on_sc=False)

    @functools.partial(
        pl.kernel,
        compiler_params=cp,
        out_type=[jax.ShapeDtypeStruct((NC, N, HD), jnp.bfloat16),
                  jax.ShapeDtypeStruct((NC, ND, HD), jnp.bfloat16)],
        mesh=mesh,
        scratch_types=[
            pltpu.VMEM((NBLK, BLK), jnp.int32),
            pltpu.VMEM((NBLK, BLK), jnp.int32),
            pltpu.VMEM((NBLK, BLK), jnp.int32),
            pltpu.VMEM((R * HD,), jnp.bfloat16),
            pltpu.VMEM((2 * DH,), jnp.float32),
            pltpu.VMEM((2, BLK, HD), jnp.bfloat16),
            pltpu.VMEM((2, BLK, HD), jnp.bfloat16),
            pltpu.VMEM((2, BLK, HD), jnp.bfloat16),
            pltpu.VMEM((2, BLK, HD), jnp.bfloat16),
            pltpu.VMEM((2, BLK, HD), jnp.bfloat16),
            pltpu.VMEM((2, BLK), jnp.int32),
            pltpu.VMEM_SHARED((N, HD), jnp.bfloat16),
            pltpu.VMEM_SHARED((ND, HD), jnp.bfloat16),
            pltpu.SemaphoreType.DMA,
            pltpu.SemaphoreType.DMA,
            pltpu.SemaphoreType.DMA,
            pltpu.SemaphoreType.DMA,
        ],
    )
    def k(src_h, dst_h, typ_h, q_h, k_h, v_h, rel_h, zn_h, zd_h, outn_h, outd_h,
          srci, dsti, typi, relv, swp, qg, kg, vg, stgn, stgd, drow, accn, accd,
          gs0, gs1, ss0, ss1):
        cid = lax.axis_index("c")
        sid = lax.axis_index("s")
        wid = cid * NS + sid
        gsem = (gs0, gs1)
        ssem = (ss0, ss1)

        pltpu.sync_copy(rel_h, relv)
        pltpu.sync_copy(src_h.at[wid], srci)
        pltpu.sync_copy(dst_h.at[wid], dsti)
        pltpu.sync_copy(typ_h.at[wid], typi)

        for t in range(ZPT):
            c = sid + t * NS

            @pl.when(c < NZC)
            def _():
                pltpu.sync_copy(zn_h, accn.at[pl.ds(c * ZCH, ZCH)])

        pltpu.sync_copy(zd_h, accd.at[pl.ds(sid * DPT, DPT)])
        plsc.subcore_barrier()

        zero = jnp.zeros((DH,), jnp.float32)
        lane = lax.broadcasted_iota(jnp.int32, (DH,), 0)
        lane7 = lane & 7
        lhalf = lane >> 3

        def issue_gathers(b, u):
            pltpu.async_copy(q_h.at[dsti.at[b]], qg.at[u], gsem[u])
            pltpu.async_copy(k_h.at[srci.at[b]], kg.at[u], gsem[u])
            pltpu.async_copy(v_h.at[srci.at[b]], vg.at[u], gsem[u])

        def drain_gathers(b, u):
            pltpu.make_async_copy(q_h.at[dsti.at[b]], qg.at[u], gsem[u]).wait()
            pltpu.make_async_copy(k_h.at[srci.at[b]], kg.at[u], gsem[u]).wait()
            pltpu.make_async_copy(v_h.at[srci.at[b]], vg.at[u], gsem[u]).wait()

        def issue_scatters(b, u):
            pltpu.async_copy(stgn.at[u], accn.at[dsti.at[b]], ssem[u], add=True)
            pltpu.async_copy(stgd.at[u], accd.at[drow.at[u]], ssem[u], add=True)

        def drain_scatters(b, u):
            pltpu.make_async_copy(stgn.at[u], accn.at[dsti.at[b]],
                                  ssem[u]).wait()
            pltpu.make_async_copy(stgd.at[u], accd.at[drow.at[u]],
                                  ssem[u]).wait()

        zb32 = jnp.zeros((2 * DH,), jnp.bfloat16)

        def compute(b, u):
            @pl.loop(0, BLK // DH)
            def _(g):
                tv = typi[b, pl.ds(g * DH, DH)]
                dv = dsti[b, pl.ds(g * DH, DH)]
                drow[u, pl.ds(g * DH, DH)] = dv >> 4
                for j in range(DH):
                    e = g * DH + j
                    rb = tv[j] * HD
                    d = dv[j]
                    acc0 = zero
                    acc1 = zero
                    for t in range(NH // 2):
                        qa, qb = plsc.unpack(
                            qg[u, e, pl.ds(t * 2 * DH, 2 * DH)],
                            format=plsc.PackFormat.INTERLEAVED)
                        ka, kb = plsc.unpack(
                            kg[u, e, pl.ds(t * 2 * DH, 2 * DH)],
                            format=plsc.PackFormat.INTERLEAVED)
                        ra, rc = plsc.unpack(
                            relv[pl.ds(rb + t * 2 * DH, 2 * DH)],
                            format=plsc.PackFormat.INTERLEAVED)
                        acc0 = acc0 + qa * (ka + ra)
                        acc1 = acc1 + qb * (kb + rc)
                    acc = acc0 + acc1
                    swp[pl.ds(0, DH)] = acc
                    swp[pl.ds(DH, DH)] = acc
                    sv = acc + swp[pl.ds(NH, DH)]
                    ev = jnp.exp(jnp.minimum(sv * INV_SCALE, 60.0))
                    for i in range(NH // 2):
                        esc = plsc.pack(zero + ev[2 * i], zero + ev[2 * i + 1],
                                        format=plsc.PackFormat.INTERLEAVED)
                        stgn[u, e, pl.ds(i * 2 * DH, 2 * DH)] = (
                            vg[u, e, pl.ds(i * 2 * DH, 2 * DH)] * esc)
                    dmask = jnp.where(lhalf == (d & 1), 1.0, 0.0)
                    dvreg = ev * dmask
                    p = (d & 15) >> 1
                    pp = p >> 1
                    packed = plsc.pack(
                        jnp.where((p & 1) == 0, dvreg, zero),
                        jnp.where((p & 1) == 1, dvreg, zero),
                        format=plsc.PackFormat.INTERLEAVED)
                    for i in range(NH // 2):
                        stgd[u, e, pl.ds(i * 2 * DH, 2 * DH)] = jnp.where(
                            pp == i, packed, zb32)

        issue_gathers(0, 0)

        @pl.loop(0, NBLK // 2)
        def _(pr):
            b0 = 2 * pr
            b1 = 2 * pr + 1
            drain_gathers(b0, 0)
            issue_gathers(b1, 1)

            @pl.when(pr > 0)
            def _():
                drain_scatters(b0 - 2, 0)

            compute(b0, 0)
            issue_scatters(b0, 0)

            drain_gathers(b1, 1)

            @pl.when(b1 + 1 < NBLK)
            def _():
                issue_gathers(b1 + 1, 0)

            @pl.when(pr > 0)
            def _():
                drain_scatters(b1 - 2, 1)

            compute(b1, 1)
            issue_scatters(b1, 1)

        tb = NBLK - 1
        drain_gathers(tb, 0)
        drain_scatters(tb - 2, 0)
        compute(tb, 0)
        issue_scatters(tb, 0)
        drain_scatters(tb, 0)
        drain_scatters(tb - 1, 1)

        plsc.subcore_barrier()
        for t in range(ZPT):
            c = sid + t * NS

            @pl.when(c < NZC)
            def _():
                rows = pl.ds(c * ZCH, ZCH)
                pltpu.sync_copy(accn.at[rows], outn_h.at[cid].at[rows])

        drows = pl.ds(sid * DPT, DPT)
        pltpu.sync_copy(accd.at[drows], outd_h.at[cid].at[drows])

    return k(src3, dst3, typ3, q2, k2, v2, rel1, zn, zd)


def _qkv(nf, qe, rel, wqt, wqb, bq, wkt, wkb, bk, wv, bv):
    def body(nf_r, qe_r, rel_r, wqt_r, wqb_r, bq_r, wkt_r, wkb_r, bk_r,
             wv_r, bv_r, q_r, k_r, v_r, rel_o):
        nfv = nf_r[...]
        qev = qe_r[...]
        oi = lax.broadcasted_iota(jnp.int32, (HD, HD), 0)
        ci = lax.broadcasted_iota(jnp.int32, (HD, HD), 1)
        vsrc = (2 * (ci // 32) + (ci % 2)) * DH + (ci % 32) // 2
        perm_v = jnp.where(oi == vsrc, 1.0, 0.0).astype(jnp.float32)
        t_ = ci // 32
        m_ = (ci % 32) // 2
        par_ = ci % 2
        qksrc = (m_ % NH) * DH + NH * par_ + 2 * t_ + m_ // NH
        perm_qk = jnp.where(oi == qksrc, 1.0, 0.0).astype(jnp.float32)
        qb = jnp.dot(qev, wqb_r[...], preferred_element_type=jnp.float32) + bq_r[...]
        kb = jnp.dot(qev, wkb_r[...], preferred_element_type=jnp.float32) + bk_r[...]
        q = jnp.dot(nfv, wqt_r[...], preferred_element_type=jnp.float32) + qb
        k = jnp.dot(nfv, wkt_r[...], preferred_element_type=jnp.float32) + kb
        v = jnp.dot(nfv, wv_r[...], preferred_element_type=jnp.float32) + bv_r[...]
        q_r[...] = jnp.dot(q, perm_qk, preferred_element_type=jnp.float32).astype(jnp.bfloat16)
        k_r[...] = jnp.dot(k, perm_qk, preferred_element_type=jnp.float32).astype(jnp.bfloat16)
        v_r[...] = jnp.dot(v, perm_v, preferred_element_type=jnp.float32).astype(jnp.bfloat16)
        rel_o[...] = jnp.dot(rel_r[...], perm_qk,
                             preferred_element_type=jnp.float32).astype(jnp.bfloat16)

    return pl.pallas_call(
        body,
        out_shape=[jax.ShapeDtypeStruct((N, HD), jnp.bfloat16)] * 3
        + [jax.ShapeDtypeStruct((R, HD), jnp.bfloat16)],
    )(nf, qe, rel, wqt, wqb, bq, wkt, wkb, bk, wv, bv)


def _layernorm(x, g, b, eps=1e-5):
    m = jnp.mean(x, axis=-1, keepdims=True)
    v = jnp.mean((x - m) * (x - m), axis=-1, keepdims=True)
    return (x - m) / jnp.sqrt(v + eps) * g + b


def _post(nf, num0, num1, den0, den1, Wo, bo, ln1_g, ln1_b,
          Wf1, bf1, Wf2, bf2, ln2_g, ln2_b):
    RB = 2000
    G = N // RB

    def body(nf_r, n0_r, n1_r, d0_r, d1_r, wo_r, bo_r, g1_r, b1_r,
             wf1_r, bf1_r, wf2_r, bf2_r, g2_r, b2_r, out_r):
        nsum = (n0_r[...].astype(jnp.float32) + n1_r[...].astype(jnp.float32))
        ci = lax.broadcasted_iota(jnp.int32, (HD, HD), 0)
        oi = lax.broadcasted_iota(jnp.int32, (HD, HD), 1)
        o_of_c = (2 * (ci // 32) + (ci % 2)) * DH + (ci % 32) // 2
        perm = jnp.where(o_of_c == oi, 1.0, 0.0).astype(jnp.float32)
        num = jnp.dot(nsum, perm, preferred_element_type=jnp.float32)
        den = d0_r[...] + d1_r[...] + 1e-8
        col = lax.broadcasted_iota(jnp.int32, (NH, HD), 1)
        row = lax.broadcasted_iota(jnp.int32, (NH, HD), 0)
        expand = jnp.where(col // DH == row, 1.0, 0.0).astype(jnp.float32)
        denr = jnp.dot(den, expand, preferred_element_type=jnp.float32)
        agg = num / denr
        attn = jnp.dot(agg, wo_r[...], preferred_element_type=jnp.float32) + bo_r[...]
        x = _layernorm(nf_r[...] + attn, g1_r[...], b1_r[...])
        hpre = jnp.dot(x, wf1_r[...], preferred_element_type=jnp.float32) + bf1_r[...]
        hact = 0.5 * hpre * (1.0 + lax.erf(hpre * (2.0 ** -0.5)))
        h2 = jnp.dot(hact, wf2_r[...], preferred_element_type=jnp.float32) + bf2_r[...]
        out_r[...] = _layernorm(x + h2, g2_r[...], b2_r[...])

    rowspec = lambda w: pl.BlockSpec((RB, w), lambda i: (i, 0))
    full = lambda a: pl.BlockSpec(a.shape, lambda i: tuple(0 for _ in a.shape))
    return pl.pallas_call(
        body,
        grid=(G,),
        in_specs=[rowspec(HD), rowspec(HD), rowspec(HD), rowspec(NH), rowspec(NH),
                  full(Wo), full(bo), full(ln1_g), full(ln1_b),
                  full(Wf1), full(bf1), full(Wf2), full(bf2),
                  full(ln2_g), full(ln2_b)],
        out_specs=pl.BlockSpec((RB, HD), lambda i: (i, 0)),
        out_shape=jax.ShapeDtypeStruct((N, HD), jnp.float32),
    )(nf, num0, num1, den0, den1, Wo, bo, ln1_g, ln1_b,
      Wf1, bf1, Wf2, bf2, ln2_g, ln2_b)


def _rel_update(rel, Wh2t, bh2t, Wt2h, bt2h, Wh2h, bh2h, Wt2t, bt2t,
                Wc1, Wc2, Wc3, Wc4, bc, lnr_g, lnr_b):
    def body(rel_r, wa_r, ba_r, wb_r, bb_r, wc_r, bcc_r, wd_r, bd_r,
             w1_r, w2_r, w3_r, w4_r, bc_r, g_r, b_r, out_r):
        r = rel_r[...]
        i1 = jnp.dot(r, wa_r[...], preferred_element_type=jnp.float32) + ba_r[...]
        i2 = jnp.dot(r, wb_r[...], preferred_element_type=jnp.float32) + bb_r[...]
        i3 = jnp.dot(r, wc_r[...], preferred_element_type=jnp.float32) + bcc_r[...]
        i4 = jnp.dot(r, wd_r[...], preferred_element_type=jnp.float32) + bd_r[...]
        comb = (jnp.dot(i1, w1_r[...], preferred_element_type=jnp.float32)
                + jnp.dot(i2, w2_r[...], preferred_element_type=jnp.float32)
                + jnp.dot(i3, w3_r[...], preferred_element_type=jnp.float32)
                + jnp.dot(i4, w4_r[...], preferred_element_type=jnp.float32)
                + bc_r[...])
        out_r[...] = _layernorm(r + comb, g_r[...], b_r[...])

    return pl.pallas_call(
        body,
        out_shape=jax.ShapeDtypeStruct((R, HD), jnp.float32),
    )(rel, Wh2t, bh2t, Wt2h, bt2h, Wh2h, bh2h, Wt2t, bt2t,
      Wc1, Wc2, Wc3, Wc4, bc, lnr_g, lnr_b)


def kernel(node_features, query_embedding, edge_index, edge_type,
           relation_embeddings, Wq, bq, Wk, bk, Wv, bv, Wo, bo, ln1_g, ln1_b,
           Wf1, bf1, Wf2, bf2, ln2_g, ln2_b, Wh2t, bh2t, Wt2h, bt2h,
           Wh2h, bh2h, Wt2t, bt2t, Wc, bc, lnr_g, lnr_b):
    r1 = lambda a: a.reshape(1, -1)
    q, k, v, reli = _qkv(node_features, query_embedding, relation_embeddings,
                         Wq[:HD], Wq[HD:], r1(bq), Wk[:HD], Wk[HD:], r1(bk),
                         Wv, r1(bv))

    part_n, part_d = _edge_pass(
        edge_index[0].reshape(NW, NBLK, BLK),
        edge_index[1].reshape(NW, NBLK, BLK),
        edge_type.reshape(NW, NBLK, BLK),
        q, k, v,
        reli.reshape(R * HD),
        jnp.zeros((ZCH, HD), jnp.bfloat16),
        jnp.zeros((DPT, HD), jnp.bfloat16))

    dg = part_d.astype(jnp.float32).reshape(NC, ND, HD // 32, 16, 2)
    dg = dg.transpose(0, 1, 2, 4, 3).reshape(NC, ND * DH, NH)
    den = dg[:, :N]
    x = _post(node_features, part_n[0], part_n[1], den[0], den[1],
              Wo, r1(bo), r1(ln1_g), r1(ln1_b),
              Wf1, r1(bf1), Wf2, r1(bf2), r1(ln2_g), r1(ln2_b))

    rel = _rel_update(relation_embeddings, Wh2t, r1(bh2t), Wt2h, r1(bt2h),
                      Wh2h, r1(bh2h), Wt2t, r1(bt2t),
                      Wc[:HD], Wc[HD:2 * HD], Wc[2 * HD:3 * HD], Wc[3 * HD:],
                      r1(bc), r1(lnr_g), r1(lnr_b))
    return (x, rel)
